# baseline (device time: 51379 ns/iter reference)
import jax
import jax.numpy as jnp
from jax import lax
from jax.experimental import pallas as pl
from jax.experimental.pallas import tpu as pltpu

N_DEV = 4
SQ = 256
D = 1024
SKV = 4096
DH = 128
HQ_SHARD = 8
KV_SHARD = 2
QC = D // 4
SCALE = 0.08838834764831843


def kernel(x, Wq, Wo, K_ext, V_ext):
    my_pos = lax.axis_index("i").astype(jnp.int32)

    def body(pos_ref, x_ref, wq_ref, wo_ref, k_any, v_any, out_ref,
             kv_ref, vv_ref, attn_ref, acc_ref, recv1_ref, recv2_ref,
             kv_sems, send_sems, recv_sems):
        my = pos_ref[0]
        b0 = lax.rem(my, 2)
        b1 = my // 2
        p1 = jnp.bitwise_xor(my, 1)
        p2 = jnp.bitwise_xor(my, 2)
        half_lo = 2 * b0
        oth_lo = 2 * (1 - b0)
        q_my = half_lo + b1
        q_oth = half_lo + (1 - b1)

        kv_copies = []
        for g in range(KV_SHARD):
            hd = 2 * my + g
            for j, (src, dst) in enumerate(((k_any, kv_ref), (v_any, vv_ref))):
                cp = pltpu.make_async_copy(
                    src.at[0, :, hd, :], dst.at[g], kv_sems.at[2 * g + j])
                cp.start()
                kv_copies.append(cp)

        barrier = pltpu.get_barrier_semaphore()
        for nbr in (p1, p2):
            pl.semaphore_signal(barrier, inc=1, device_id=(nbr,),
                                device_id_type=pl.DeviceIdType.MESH)
        pl.semaphore_wait(barrier, 2)

        q = jnp.dot(x_ref[0].astype(jnp.bfloat16),
                    wq_ref[:, :].astype(jnp.bfloat16),
                    preferred_element_type=jnp.float32)

        for cp in kv_copies:
            cp.wait()

        for h in range(HQ_SHARD):
            g = h // 4
            q_h = q[:, h * DH:(h + 1) * DH].astype(jnp.bfloat16)
            k_h = kv_ref[g].astype(jnp.bfloat16)
            v_h = vv_ref[g].astype(jnp.bfloat16)
            s = lax.dot_general(q_h, k_h, (((1,), (1,)), ((), ())),
                                preferred_element_type=jnp.float32) * SCALE
            m = jnp.max(s, axis=1, keepdims=True)
            p = jnp.exp(s - m)
            l = jnp.sum(p, axis=1, keepdims=True)
            o = jnp.dot(p.astype(jnp.bfloat16), v_h,
                        preferred_element_type=jnp.float32) / l
            attn_ref[:, h * DH:(h + 1) * DH] = o

        partial = jnp.dot(attn_ref[:, :].astype(jnp.bfloat16),
                          wo_ref[:, :].astype(jnp.bfloat16),
                          preferred_element_type=jnp.float32)
        for qq in range(N_DEV):
            acc_ref[qq] = partial[:, qq * QC:(qq + 1) * QC]

        rs1 = pltpu.make_async_remote_copy(
            src_ref=acc_ref.at[pl.ds(oth_lo, 2)],
            dst_ref=recv1_ref,
            send_sem=send_sems.at[0],
            recv_sem=recv_sems.at[0],
            device_id=(p1,),
            device_id_type=pl.DeviceIdType.MESH,
        )
        rs1.start()
        rs1.wait()
        acc_ref[pl.ds(half_lo, 2)] = acc_ref[pl.ds(half_lo, 2)] + recv1_ref[:, :, :]

        rs2 = pltpu.make_async_remote_copy(
            src_ref=acc_ref.at[pl.ds(q_oth, 1)],
            dst_ref=recv2_ref,
            send_sem=send_sems.at[1],
            recv_sem=recv_sems.at[1],
            device_id=(p2,),
            device_id_type=pl.DeviceIdType.MESH,
        )
        rs2.start()
        rs2.wait()
        acc_ref[pl.ds(q_my, 1)] = acc_ref[pl.ds(q_my, 1)] + recv2_ref[:, :, :]

        ag3 = pltpu.make_async_remote_copy(
            src_ref=acc_ref.at[pl.ds(q_my, 1)],
            dst_ref=acc_ref.at[pl.ds(q_my, 1)],
            send_sem=send_sems.at[2],
            recv_sem=recv_sems.at[2],
            device_id=(p2,),
            device_id_type=pl.DeviceIdType.MESH,
        )
        ag3.start()
        ag3.wait()

        ag4 = pltpu.make_async_remote_copy(
            src_ref=acc_ref.at[pl.ds(half_lo, 2)],
            dst_ref=acc_ref.at[pl.ds(half_lo, 2)],
            send_sem=send_sems.at[3],
            recv_sem=recv_sems.at[3],
            device_id=(p1,),
            device_id_type=pl.DeviceIdType.MESH,
        )
        ag4.start()
        ag4.wait()

        for qq in range(N_DEV):
            out_ref[0, :, qq * QC:(qq + 1) * QC] = acc_ref[qq]

    grid_spec = pltpu.PrefetchScalarGridSpec(
        num_scalar_prefetch=1,
        grid=(1,),
        in_specs=[
            pl.BlockSpec((1, SQ, D), lambda i, m: (0, 0, 0)),
            pl.BlockSpec((D, D), lambda i, m: (0, 0)),
            pl.BlockSpec((D, D), lambda i, m: (0, 0)),
            pl.BlockSpec(memory_space=pl.ANY),
            pl.BlockSpec(memory_space=pl.ANY),
        ],
        out_specs=pl.BlockSpec((1, SQ, D), lambda i, m: (0, 0, 0)),
        scratch_shapes=[
            pltpu.VMEM((KV_SHARD, SKV, DH), jnp.float32),
            pltpu.VMEM((KV_SHARD, SKV, DH), jnp.float32),
            pltpu.VMEM((SQ, D), jnp.float32),
            pltpu.VMEM((N_DEV, SQ, QC), jnp.float32),
            pltpu.VMEM((2, SQ, QC), jnp.float32),
            pltpu.VMEM((1, SQ, QC), jnp.float32),
            pltpu.SemaphoreType.DMA((4,)),
            pltpu.SemaphoreType.DMA((N_DEV,)),
            pltpu.SemaphoreType.DMA((N_DEV,)),
        ],
    )
    return pl.pallas_call(
        body,
        grid_spec=grid_spec,
        out_shape=jax.ShapeDtypeStruct((1, SQ, D), jnp.float32),
        compiler_params=pltpu.CompilerParams(collective_id=0),
    )(my_pos.reshape(1), x, Wq, Wo, K_ext, V_ext)


# device time: 50361 ns/iter; 1.0202x vs baseline; 1.0202x over previous
import jax
import jax.numpy as jnp
from jax import lax
from jax.experimental import pallas as pl
from jax.experimental.pallas import tpu as pltpu

N_DEV = 4
SQ = 256
SQH = SQ // 2
D = 1024
SKV = 4096
DH = 128
HQ_SHARD = 8
KV_SHARD = 2
QC = D // 4
SCALE = 0.08838834764831843


def kernel(x, Wq, Wo, K_ext, V_ext):
    my_pos = lax.axis_index("i").astype(jnp.int32)

    def body(pos_ref, x_ref, wq_ref, wo_ref, k_any, v_any, out_ref,
             kv_ref, vv_ref, attn_ref, acc_ref, recv1_ref, recv2_ref,
             kv_sems, send_sems, recv_sems):
        my = pos_ref[0]
        b0 = lax.rem(my, 2)
        b1 = my // 2
        p1 = jnp.bitwise_xor(my, 1)
        p2 = jnp.bitwise_xor(my, 2)
        half_lo = 2 * b0
        oth_lo = 2 * (1 - b0)
        q_my = half_lo + b1
        q_oth = half_lo + (1 - b1)

        kv_copies = []
        for g in range(KV_SHARD):
            hd = 2 * my + g
            for j, (src, dst) in enumerate(((k_any, kv_ref), (v_any, vv_ref))):
                cp = pltpu.make_async_copy(
                    src.at[0, :, hd, :], dst.at[g], kv_sems.at[2 * g + j])
                cp.start()
                kv_copies.append(cp)

        barrier = pltpu.get_barrier_semaphore()
        for nbr in (p1, p2):
            pl.semaphore_signal(barrier, inc=1, device_id=(nbr,),
                                device_id_type=pl.DeviceIdType.MESH)
        pl.semaphore_wait(barrier, 2)

        q = jnp.dot(x_ref[0].astype(jnp.bfloat16),
                    wq_ref[:, :].astype(jnp.bfloat16),
                    preferred_element_type=jnp.float32)

        for cp in kv_copies:
            cp.wait()

        def compute_half(r):
            base = r * SQH
            for h in range(HQ_SHARD):
                g = h // 4
                q_h = q[base:base + SQH,
                        h * DH:(h + 1) * DH].astype(jnp.bfloat16)
                k_h = kv_ref[g].astype(jnp.bfloat16)
                v_h = vv_ref[g].astype(jnp.bfloat16)
                s = lax.dot_general(q_h, k_h, (((1,), (1,)), ((), ())),
                                    preferred_element_type=jnp.float32) * SCALE
                m = jnp.max(s, axis=1, keepdims=True)
                p = jnp.exp(s - m)
                l = jnp.sum(p, axis=1, keepdims=True)
                o = jnp.dot(p.astype(jnp.bfloat16), v_h,
                            preferred_element_type=jnp.float32) / l
                attn_ref[base:base + SQH, h * DH:(h + 1) * DH] = o
            partial = jnp.dot(attn_ref[base:base + SQH, :].astype(jnp.bfloat16),
                              wo_ref[:, :].astype(jnp.bfloat16),
                              preferred_element_type=jnp.float32)
            for qq in range(N_DEV):
                acc_ref[r, qq] = partial[:, qq * QC:(qq + 1) * QC]

        def rc(r, step, src, dst, peer):
            return pltpu.make_async_remote_copy(
                src_ref=src, dst_ref=dst,
                send_sem=send_sems.at[r, step],
                recv_sem=recv_sems.at[r, step],
                device_id=(peer,),
                device_id_type=pl.DeviceIdType.MESH,
            )

        def rs1(r):
            return rc(r, 0, acc_ref.at[r, pl.ds(oth_lo, 2)],
                      recv1_ref.at[r], p1)

        def add1(r):
            acc_ref[r, pl.ds(half_lo, 2)] = (
                acc_ref[r, pl.ds(half_lo, 2)] + recv1_ref[r])

        def rs2(r):
            return rc(r, 1, acc_ref.at[r, pl.ds(q_oth, 1)],
                      recv2_ref.at[r], p2)

        def add2(r):
            acc_ref[r, pl.ds(q_my, 1)] = (
                acc_ref[r, pl.ds(q_my, 1)] + recv2_ref[r])

        def ag3(r):
            return rc(r, 2, acc_ref.at[r, pl.ds(q_my, 1)],
                      acc_ref.at[r, pl.ds(q_my, 1)], p2)

        def ag4(r):
            return rc(r, 3, acc_ref.at[r, pl.ds(half_lo, 2)],
                      acc_ref.at[r, pl.ds(half_lo, 2)], p1)

        compute_half(0)
        rs1_0 = rs1(0)
        rs1_0.start()
        compute_half(1)
        rs1_1 = rs1(1)
        rs1_1.start()
        rs1_0.wait()
        add1(0)
        rs2_0 = rs2(0)
        rs2_0.start()
        rs1_1.wait()
        add1(1)
        rs2_0.wait()
        add2(0)
        ag3_0 = ag3(0)
        ag3_0.start()
        rs2_1 = rs2(1)
        rs2_1.start()
        ag3_0.wait()
        ag4_0 = ag4(0)
        ag4_0.start()
        rs2_1.wait()
        add2(1)
        ag3_1 = ag3(1)
        ag3_1.start()
        ag4_0.wait()
        ag3_1.wait()
        ag4_1 = ag4(1)
        ag4_1.start()
        ag4_1.wait()

        for r in range(2):
            for qq in range(N_DEV):
                out_ref[0, r * SQH:(r + 1) * SQH,
                        qq * QC:(qq + 1) * QC] = acc_ref[r, qq]

    grid_spec = pltpu.PrefetchScalarGridSpec(
        num_scalar_prefetch=1,
        grid=(1,),
        in_specs=[
            pl.BlockSpec((1, SQ, D), lambda i, m: (0, 0, 0)),
            pl.BlockSpec((D, D), lambda i, m: (0, 0)),
            pl.BlockSpec((D, D), lambda i, m: (0, 0)),
            pl.BlockSpec(memory_space=pl.ANY),
            pl.BlockSpec(memory_space=pl.ANY),
        ],
        out_specs=pl.BlockSpec((1, SQ, D), lambda i, m: (0, 0, 0)),
        scratch_shapes=[
            pltpu.VMEM((KV_SHARD, SKV, DH), jnp.float32),
            pltpu.VMEM((KV_SHARD, SKV, DH), jnp.float32),
            pltpu.VMEM((SQ, D), jnp.float32),
            pltpu.VMEM((2, N_DEV, SQH, QC), jnp.float32),
            pltpu.VMEM((2, 2, SQH, QC), jnp.float32),
            pltpu.VMEM((2, 1, SQH, QC), jnp.float32),
            pltpu.SemaphoreType.DMA((4,)),
            pltpu.SemaphoreType.DMA((2, N_DEV)),
            pltpu.SemaphoreType.DMA((2, N_DEV)),
        ],
    )
    return pl.pallas_call(
        body,
        grid_spec=grid_spec,
        out_shape=jax.ShapeDtypeStruct((1, SQ, D), jnp.float32),
        compiler_params=pltpu.CompilerParams(collective_id=0),
    )(my_pos.reshape(1), x, Wq, Wo, K_ext, V_ext)


# device time: 45237 ns/iter; 1.1358x vs baseline; 1.1133x over previous
import jax
import jax.numpy as jnp
from jax import lax
from jax.experimental import pallas as pl
from jax.experimental.pallas import tpu as pltpu

N_DEV = 4
SQ = 256
SQH = SQ // 2
D = 1024
SKV = 4096
DH = 128
HQ_SHARD = 8
KV_SHARD = 2
QC = D // 4
SCALE = 0.08838834764831843


def kernel(x, Wq, Wo, K_ext, V_ext):
    my_pos = lax.axis_index("i").astype(jnp.int32)

    def body(pos_ref, x_ref, wq_ref, wo_ref, k_any, v_any, out_ref,
             kv_ref, vv_ref, attn_ref, acc_ref, recv1_ref, recv2_ref,
             kv_sems, send_sems, recv_sems):
        my = pos_ref[0]
        b0 = lax.rem(my, 2)
        b1 = my // 2
        p1 = jnp.bitwise_xor(my, 1)
        p2 = jnp.bitwise_xor(my, 2)
        half_lo = 2 * b0
        oth_lo = 2 * (1 - b0)
        q_my = half_lo + b1
        q_oth = half_lo + (1 - b1)

        kv_copies = []
        for g in range(KV_SHARD):
            hd = 2 * my + g
            for j, (src, dst) in enumerate(((k_any, kv_ref), (v_any, vv_ref))):
                cp = pltpu.make_async_copy(
                    src.at[0, :, hd, :], dst.at[g], kv_sems.at[2 * g + j])
                cp.start()
                kv_copies.append(cp)

        barrier = pltpu.get_barrier_semaphore()
        for nbr in (p1, p2):
            pl.semaphore_signal(barrier, inc=1, device_id=(nbr,),
                                device_id_type=pl.DeviceIdType.MESH)
        pl.semaphore_wait(barrier, 2)

        q = jnp.dot(x_ref[0].astype(jnp.bfloat16),
                    wq_ref[:, :].astype(jnp.bfloat16),
                    preferred_element_type=jnp.float32)
        qs = (q * SCALE).astype(jnp.bfloat16)

        for cp in kv_copies:
            cp.wait()
        kb = kv_ref[:, :, :].astype(jnp.bfloat16)
        vb = vv_ref[:, :, :].astype(jnp.bfloat16)

        def compute_half(r):
            base = r * SQH
            for h in range(HQ_SHARD):
                g = h // 4
                q_h = qs[base:base + SQH, h * DH:(h + 1) * DH]
                k_h = kb[g]
                v_h = vb[g]
                s = lax.dot_general(q_h, k_h, (((1,), (1,)), ((), ())),
                                    preferred_element_type=jnp.float32)
                p = jnp.exp(s)
                l = jnp.sum(p, axis=1, keepdims=True)
                o = jnp.dot(p.astype(jnp.bfloat16), v_h,
                            preferred_element_type=jnp.float32) / l
                attn_ref[base:base + SQH, h * DH:(h + 1) * DH] = o
            partial = jnp.dot(attn_ref[base:base + SQH, :].astype(jnp.bfloat16),
                              wo_ref[:, :].astype(jnp.bfloat16),
                              preferred_element_type=jnp.float32)
            for qq in range(N_DEV):
                acc_ref[r, qq] = partial[:, qq * QC:(qq + 1) * QC]

        def rc(r, step, src, dst, peer):
            return pltpu.make_async_remote_copy(
                src_ref=src, dst_ref=dst,
                send_sem=send_sems.at[r, step],
                recv_sem=recv_sems.at[r, step],
                device_id=(peer,),
                device_id_type=pl.DeviceIdType.MESH,
            )

        def rs1(r):
            return rc(r, 0, acc_ref.at[r, pl.ds(oth_lo, 2)],
                      recv1_ref.at[r], p1)

        def add1(r):
            acc_ref[r, pl.ds(half_lo, 2)] = (
                acc_ref[r, pl.ds(half_lo, 2)] + recv1_ref[r])

        def rs2(r):
            return rc(r, 1, acc_ref.at[r, pl.ds(q_oth, 1)],
                      recv2_ref.at[r], p2)

        def add2(r):
            acc_ref[r, pl.ds(q_my, 1)] = (
                acc_ref[r, pl.ds(q_my, 1)] + recv2_ref[r])

        def ag3(r):
            return rc(r, 2, acc_ref.at[r, pl.ds(q_my, 1)],
                      acc_ref.at[r, pl.ds(q_my, 1)], p2)

        def ag4(r):
            return rc(r, 3, acc_ref.at[r, pl.ds(half_lo, 2)],
                      acc_ref.at[r, pl.ds(half_lo, 2)], p1)

        compute_half(0)
        rs1_0 = rs1(0)
        rs1_0.start()
        compute_half(1)
        rs1_1 = rs1(1)
        rs1_1.start()
        rs1_0.wait()
        add1(0)
        rs2_0 = rs2(0)
        rs2_0.start()
        rs1_1.wait()
        add1(1)
        rs2_0.wait()
        add2(0)
        ag3_0 = ag3(0)
        ag3_0.start()
        rs2_1 = rs2(1)
        rs2_1.start()
        ag3_0.wait()
        ag4_0 = ag4(0)
        ag4_0.start()
        rs2_1.wait()
        add2(1)
        ag3_1 = ag3(1)
        ag3_1.start()
        ag4_0.wait()
        ag3_1.wait()
        ag4_1 = ag4(1)
        ag4_1.start()
        ag4_1.wait()

        for r in range(2):
            for qq in range(N_DEV):
                out_ref[0, r * SQH:(r + 1) * SQH,
                        qq * QC:(qq + 1) * QC] = acc_ref[r, qq]

    grid_spec = pltpu.PrefetchScalarGridSpec(
        num_scalar_prefetch=1,
        grid=(1,),
        in_specs=[
            pl.BlockSpec((1, SQ, D), lambda i, m: (0, 0, 0)),
            pl.BlockSpec((D, D), lambda i, m: (0, 0)),
            pl.BlockSpec((D, D), lambda i, m: (0, 0)),
            pl.BlockSpec(memory_space=pl.ANY),
            pl.BlockSpec(memory_space=pl.ANY),
        ],
        out_specs=pl.BlockSpec((1, SQ, D), lambda i, m: (0, 0, 0)),
        scratch_shapes=[
            pltpu.VMEM((KV_SHARD, SKV, DH), jnp.float32),
            pltpu.VMEM((KV_SHARD, SKV, DH), jnp.float32),
            pltpu.VMEM((SQ, D), jnp.float32),
            pltpu.VMEM((2, N_DEV, SQH, QC), jnp.float32),
            pltpu.VMEM((2, 2, SQH, QC), jnp.float32),
            pltpu.VMEM((2, 1, SQH, QC), jnp.float32),
            pltpu.SemaphoreType.DMA((4,)),
            pltpu.SemaphoreType.DMA((2, N_DEV)),
            pltpu.SemaphoreType.DMA((2, N_DEV)),
        ],
    )
    return pl.pallas_call(
        body,
        grid_spec=grid_spec,
        out_shape=jax.ShapeDtypeStruct((1, SQ, D), jnp.float32),
        compiler_params=pltpu.CompilerParams(collective_id=0),
    )(my_pos.reshape(1), x, Wq, Wo, K_ext, V_ext)


# device time: 35321 ns/iter; 1.4546x vs baseline; 1.2807x over previous
import jax
import jax.numpy as jnp
from jax import lax
from jax.experimental import pallas as pl
from jax.experimental.pallas import tpu as pltpu

N_DEV = 4
SQ = 256
SQH = SQ // 2
D = 1024
SKV = 4096
DH = 128
HQ_SHARD = 8
KV_SHARD = 2
QC = D // 4
SCALE = 0.08838834764831843


def kernel(x, Wq, Wo, K_ext, V_ext):
    my_pos = lax.axis_index("i").astype(jnp.int32)

    def body(pos_ref, x_ref, wq_ref, wo_ref, k_any, v_any, out_ref,
             kv_ref, vv_ref, attn_ref, acc_ref, stage_ref, fin_ref,
             kv_sems, send_sems, recv_sems):
        my = pos_ref[0]
        peers = [jnp.bitwise_xor(my, j + 1) for j in range(3)]

        kv_copies = []
        for g in range(KV_SHARD):
            hd = 2 * my + g
            for j, (src, dst) in enumerate(((k_any, kv_ref), (v_any, vv_ref))):
                cp = pltpu.make_async_copy(
                    src.at[0, :, hd, :], dst.at[g], kv_sems.at[2 * g + j])
                cp.start()
                kv_copies.append(cp)

        barrier = pltpu.get_barrier_semaphore()
        for nbr in peers:
            pl.semaphore_signal(barrier, inc=1, device_id=(nbr,),
                                device_id_type=pl.DeviceIdType.MESH)
        pl.semaphore_wait(barrier, 3)

        q = jnp.dot(x_ref[0].astype(jnp.bfloat16),
                    wq_ref[:, :].astype(jnp.bfloat16),
                    preferred_element_type=jnp.float32)
        qs = (q * SCALE).astype(jnp.bfloat16)

        for cp in kv_copies:
            cp.wait()
        kb = kv_ref[:, :, :].astype(jnp.bfloat16)
        vb = vv_ref[:, :, :].astype(jnp.bfloat16)

        def compute_half(r):
            base = r * SQH
            for h in range(HQ_SHARD):
                g = h // 4
                q_h = qs[base:base + SQH, h * DH:(h + 1) * DH]
                k_h = kb[g]
                v_h = vb[g]
                s = lax.dot_general(q_h, k_h, (((1,), (1,)), ((), ())),
                                    preferred_element_type=jnp.float32)
                p = jnp.exp(s)
                l = jnp.sum(p, axis=1, keepdims=True)
                o = jnp.dot(p.astype(jnp.bfloat16), v_h,
                            preferred_element_type=jnp.float32) / l
                attn_ref[base:base + SQH, h * DH:(h + 1) * DH] = o
            partial = jnp.dot(attn_ref[base:base + SQH, :].astype(jnp.bfloat16),
                              wo_ref[:, :].astype(jnp.bfloat16),
                              preferred_element_type=jnp.float32)
            for qq in range(N_DEV):
                acc_ref[r, qq] = partial[:, qq * QC:(qq + 1) * QC]

        def phase1_start(r):
            ds = []
            for j, p in enumerate(peers):
                d = pltpu.make_async_remote_copy(
                    src_ref=acc_ref.at[r, pl.ds(p, 1)],
                    dst_ref=stage_ref.at[r, pl.ds(j, 1)],
                    send_sem=send_sems.at[r, 0, j],
                    recv_sem=recv_sems.at[r, 0, j],
                    device_id=(p,),
                    device_id_type=pl.DeviceIdType.MESH,
                )
                d.start()
                ds.append(d)
            return ds

        def phase1_reduce(r, p1s):
            for j in range(3):
                pltpu.make_async_remote_copy(
                    src_ref=acc_ref.at[r, pl.ds(peers[j], 1)],
                    dst_ref=stage_ref.at[r, pl.ds(j, 1)],
                    send_sem=send_sems.at[r, 0, j],
                    recv_sem=recv_sems.at[r, 0, j],
                    device_id=(peers[j],),
                    device_id_type=pl.DeviceIdType.MESH,
                ).wait_recv()
            red = (acc_ref[r, pl.ds(my, 1)]
                   + (stage_ref[r, 0:1] + stage_ref[r, 1:2])
                   + stage_ref[r, 2:3])
            fin_ref[r, pl.ds(my, 1)] = red

        def phase2_start(r):
            ds = []
            for j, p in enumerate(peers):
                d = pltpu.make_async_remote_copy(
                    src_ref=fin_ref.at[r, pl.ds(my, 1)],
                    dst_ref=fin_ref.at[r, pl.ds(my, 1)],
                    send_sem=send_sems.at[r, 1, j],
                    recv_sem=recv_sems.at[r, 1, j],
                    device_id=(p,),
                    device_id_type=pl.DeviceIdType.MESH,
                )
                d.start()
                ds.append(d)
            return ds

        def phase2_wait(r):
            for j in range(3):
                pltpu.make_async_remote_copy(
                    src_ref=fin_ref.at[r, pl.ds(my, 1)],
                    dst_ref=fin_ref.at[r, pl.ds(peers[j], 1)],
                    send_sem=send_sems.at[r, 1, j],
                    recv_sem=recv_sems.at[r, 1, j],
                    device_id=(peers[j],),
                    device_id_type=pl.DeviceIdType.MESH,
                ).wait_recv()

        compute_half(0)
        p1s_0 = phase1_start(0)
        compute_half(1)
        p1s_1 = phase1_start(1)
        phase1_reduce(0, p1s_0)
        p2s_0 = phase2_start(0)
        phase1_reduce(1, p1s_1)
        p2s_1 = phase2_start(1)
        phase2_wait(0)
        phase2_wait(1)
        for d in p1s_0 + p1s_1 + p2s_0 + p2s_1:
            d.wait_send()

        for r in range(2):
            for qq in range(N_DEV):
                out_ref[0, r * SQH:(r + 1) * SQH,
                        qq * QC:(qq + 1) * QC] = fin_ref[r, qq]

    grid_spec = pltpu.PrefetchScalarGridSpec(
        num_scalar_prefetch=1,
        grid=(1,),
        in_specs=[
            pl.BlockSpec((1, SQ, D), lambda i, m: (0, 0, 0)),
            pl.BlockSpec((D, D), lambda i, m: (0, 0)),
            pl.BlockSpec((D, D), lambda i, m: (0, 0)),
            pl.BlockSpec(memory_space=pl.ANY),
            pl.BlockSpec(memory_space=pl.ANY),
        ],
        out_specs=pl.BlockSpec((1, SQ, D), lambda i, m: (0, 0, 0)),
        scratch_shapes=[
            pltpu.VMEM((KV_SHARD, SKV, DH), jnp.float32),
            pltpu.VMEM((KV_SHARD, SKV, DH), jnp.float32),
            pltpu.VMEM((SQ, D), jnp.float32),
            pltpu.VMEM((2, N_DEV, SQH, QC), jnp.float32),
            pltpu.VMEM((2, 3, SQH, QC), jnp.float32),
            pltpu.VMEM((2, N_DEV, SQH, QC), jnp.float32),
            pltpu.SemaphoreType.DMA((4,)),
            pltpu.SemaphoreType.DMA((2, 2, 3)),
            pltpu.SemaphoreType.DMA((2, 2, 3)),
        ],
    )
    return pl.pallas_call(
        body,
        grid_spec=grid_spec,
        out_shape=jax.ShapeDtypeStruct((1, SQ, D), jnp.float32),
        compiler_params=pltpu.CompilerParams(collective_id=0),
    )(my_pos.reshape(1), x, Wq, Wo, K_ext, V_ext)


# device time: 33725 ns/iter; 1.5235x vs baseline; 1.0473x over previous
import jax
import jax.numpy as jnp
from jax import lax
from jax.experimental import pallas as pl
from jax.experimental.pallas import tpu as pltpu

N_DEV = 4
SQ = 256
SQH = SQ // 2
D = 1024
SKV = 4096
DH = 128
HQ_SHARD = 8
KV_SHARD = 2
QC = D // 4
SCALE = 0.08838834764831843


def kernel(x, Wq, Wo, K_ext, V_ext):
    my_pos = lax.axis_index("i").astype(jnp.int32)

    def body(pos_ref, x_ref, wq_ref, wo_ref, k_any, v_any, out_ref,
             kv_ref, vv_ref, attn_ref, acc_ref, stage_ref, fin_ref,
             kv_sems, send_sems, recv_sems):
        my = pos_ref[0]
        peers = [jnp.bitwise_xor(my, j + 1) for j in range(3)]

        kv_copies = []
        for g in range(KV_SHARD):
            hd = 2 * my + g
            for j, (src, dst) in enumerate(((k_any, kv_ref), (v_any, vv_ref))):
                cp = pltpu.make_async_copy(
                    src.at[0, :, hd, :], dst.at[g], kv_sems.at[2 * g + j])
                cp.start()
                kv_copies.append(cp)

        barrier = pltpu.get_barrier_semaphore()
        for nbr in peers:
            pl.semaphore_signal(barrier, inc=1, device_id=(nbr,),
                                device_id_type=pl.DeviceIdType.MESH)
        pl.semaphore_wait(barrier, 3)

        q = jnp.dot(x_ref[0], wq_ref[:, :],
                    preferred_element_type=jnp.float32)
        qs = q * SCALE

        for cp in kv_copies:
            cp.wait()

        def compute_half(r):
            base = r * SQH
            for h in range(HQ_SHARD):
                g = h // 4
                q_h = qs[base:base + SQH, h * DH:(h + 1) * DH]
                k_h = kv_ref[g]
                v_h = vv_ref[g]
                s = lax.dot_general(q_h, k_h, (((1,), (1,)), ((), ())),
                                    preferred_element_type=jnp.float32)
                p = jnp.exp(s)
                l = jnp.sum(p, axis=1, keepdims=True)
                o = jnp.dot(p, v_h, preferred_element_type=jnp.float32) / l
                attn_ref[base:base + SQH, h * DH:(h + 1) * DH] = o
            partial = jnp.dot(attn_ref[base:base + SQH, :], wo_ref[:, :],
                              preferred_element_type=jnp.float32)
            for qq in range(N_DEV):
                acc_ref[r, qq] = partial[:, qq * QC:(qq + 1) * QC]

        def phase1_start(r):
            ds = []
            for j, p in enumerate(peers):
                d = pltpu.make_async_remote_copy(
                    src_ref=acc_ref.at[r, pl.ds(p, 1)],
                    dst_ref=stage_ref.at[r, pl.ds(j, 1)],
                    send_sem=send_sems.at[r, 0, j],
                    recv_sem=recv_sems.at[r, 0, j],
                    device_id=(p,),
                    device_id_type=pl.DeviceIdType.MESH,
                )
                d.start()
                ds.append(d)
            return ds

        def phase1_reduce(r, p1s):
            for j in range(3):
                pltpu.make_async_remote_copy(
                    src_ref=acc_ref.at[r, pl.ds(peers[j], 1)],
                    dst_ref=stage_ref.at[r, pl.ds(j, 1)],
                    send_sem=send_sems.at[r, 0, j],
                    recv_sem=recv_sems.at[r, 0, j],
                    device_id=(peers[j],),
                    device_id_type=pl.DeviceIdType.MESH,
                ).wait_recv()
            red = (acc_ref[r, pl.ds(my, 1)]
                   + (stage_ref[r, 0:1] + stage_ref[r, 1:2])
                   + stage_ref[r, 2:3])
            fin_ref[r, pl.ds(my, 1)] = red

        def phase2_start(r):
            ds = []
            for j, p in enumerate(peers):
                d = pltpu.make_async_remote_copy(
                    src_ref=fin_ref.at[r, pl.ds(my, 1)],
                    dst_ref=fin_ref.at[r, pl.ds(my, 1)],
                    send_sem=send_sems.at[r, 1, j],
                    recv_sem=recv_sems.at[r, 1, j],
                    device_id=(p,),
                    device_id_type=pl.DeviceIdType.MESH,
                )
                d.start()
                ds.append(d)
            return ds

        def phase2_wait(r):
            for j in range(3):
                pltpu.make_async_remote_copy(
                    src_ref=fin_ref.at[r, pl.ds(my, 1)],
                    dst_ref=fin_ref.at[r, pl.ds(peers[j], 1)],
                    send_sem=send_sems.at[r, 1, j],
                    recv_sem=recv_sems.at[r, 1, j],
                    device_id=(peers[j],),
                    device_id_type=pl.DeviceIdType.MESH,
                ).wait_recv()

        compute_half(0)
        p1s_0 = phase1_start(0)
        compute_half(1)
        p1s_1 = phase1_start(1)
        phase1_reduce(0, p1s_0)
        p2s_0 = phase2_start(0)
        phase1_reduce(1, p1s_1)
        p2s_1 = phase2_start(1)
        phase2_wait(0)
        phase2_wait(1)
        for d in p1s_0 + p1s_1 + p2s_0 + p2s_1:
            d.wait_send()

        for r in range(2):
            for qq in range(N_DEV):
                out_ref[0, r * SQH:(r + 1) * SQH,
                        qq * QC:(qq + 1) * QC] = fin_ref[r, qq]

    grid_spec = pltpu.PrefetchScalarGridSpec(
        num_scalar_prefetch=1,
        grid=(1,),
        in_specs=[
            pl.BlockSpec((1, SQ, D), lambda i, m: (0, 0, 0)),
            pl.BlockSpec((D, D), lambda i, m: (0, 0)),
            pl.BlockSpec((D, D), lambda i, m: (0, 0)),
            pl.BlockSpec(memory_space=pl.ANY),
            pl.BlockSpec(memory_space=pl.ANY),
        ],
        out_specs=pl.BlockSpec((1, SQ, D), lambda i, m: (0, 0, 0)),
        scratch_shapes=[
            pltpu.VMEM((KV_SHARD, SKV, DH), jnp.float32),
            pltpu.VMEM((KV_SHARD, SKV, DH), jnp.float32),
            pltpu.VMEM((SQ, D), jnp.float32),
            pltpu.VMEM((2, N_DEV, SQH, QC), jnp.float32),
            pltpu.VMEM((2, 3, SQH, QC), jnp.float32),
            pltpu.VMEM((2, N_DEV, SQH, QC), jnp.float32),
            pltpu.SemaphoreType.DMA((4,)),
            pltpu.SemaphoreType.DMA((2, 2, 3)),
            pltpu.SemaphoreType.DMA((2, 2, 3)),
        ],
    )
    return pl.pallas_call(
        body,
        grid_spec=grid_spec,
        out_shape=jax.ShapeDtypeStruct((1, SQ, D), jnp.float32),
        compiler_params=pltpu.CompilerParams(collective_id=0),
    )(my_pos.reshape(1), x, Wq, Wo, K_ext, V_ext)


# device time: 33654 ns/iter; 1.5267x vs baseline; 1.0021x over previous
import jax
import jax.numpy as jnp
from jax import lax
from jax.experimental import pallas as pl
from jax.experimental.pallas import tpu as pltpu

N_DEV = 4
SQ = 256
SQH = SQ // 2
D = 1024
SKV = 4096
DH = 128
HQ_SHARD = 8
KV_SHARD = 2
QC = D // 4
SCALE = 0.08838834764831843


def kernel(x, Wq, Wo, K_ext, V_ext):
    my_pos = lax.axis_index("i").astype(jnp.int32)

    def body(pos_ref, x_ref, wq_ref, wo_ref, k_any, v_any, out_ref,
             kv_ref, vv_ref, attn_ref, acc_ref, stage_ref, fin_ref,
             kv_sems, send_sems, recv_sems):
        my = pos_ref[0]
        peers = [jnp.bitwise_xor(my, j + 1) for j in range(3)]

        kv_copies = []
        for g in range(KV_SHARD):
            hd = 2 * my + g
            for j, (src, dst) in enumerate(((k_any, kv_ref), (v_any, vv_ref))):
                cp = pltpu.make_async_copy(
                    src.at[0, :, hd, :], dst.at[g], kv_sems.at[2 * g + j])
                cp.start()
                kv_copies.append(cp)

        barrier = pltpu.get_barrier_semaphore()
        for nbr in peers:
            pl.semaphore_signal(barrier, inc=1, device_id=(nbr,),
                                device_id_type=pl.DeviceIdType.MESH)
        pl.semaphore_wait(barrier, 3)

        q = jnp.dot(x_ref[0], wq_ref[:, :],
                    preferred_element_type=jnp.float32)
        qs = q * SCALE

        kv_waited = [False, False]

        def attn_head(r, h):
            g = h // 4
            if not kv_waited[g]:
                kv_copies[2 * g].wait()
                kv_copies[2 * g + 1].wait()
                kv_waited[g] = True
            base = r * SQH
            q_h = qs[base:base + SQH, h * DH:(h + 1) * DH]
            k_h = kv_ref[g]
            v_h = vv_ref[g]
            s = lax.dot_general(q_h, k_h, (((1,), (1,)), ((), ())),
                                preferred_element_type=jnp.float32)
            p = jnp.exp(s)
            l = jnp.sum(p, axis=1, keepdims=True)
            o = jnp.dot(p, v_h, preferred_element_type=jnp.float32) / l
            attn_ref[base:base + SQH, h * DH:(h + 1) * DH] = o

        def finish_half(r):
            base = r * SQH
            partial = jnp.dot(attn_ref[base:base + SQH, :], wo_ref[:, :],
                              preferred_element_type=jnp.float32)
            for qq in range(N_DEV):
                acc_ref[r, qq] = partial[:, qq * QC:(qq + 1) * QC]

        def phase1_start(r):
            ds = []
            for j, p in enumerate(peers):
                d = pltpu.make_async_remote_copy(
                    src_ref=acc_ref.at[r, pl.ds(p, 1)],
                    dst_ref=stage_ref.at[r, pl.ds(j, 1)],
                    send_sem=send_sems.at[r, 0, j],
                    recv_sem=recv_sems.at[r, 0, j],
                    device_id=(p,),
                    device_id_type=pl.DeviceIdType.MESH,
                )
                d.start()
                ds.append(d)
            return ds

        def phase1_reduce(r, p1s):
            for j in range(3):
                pltpu.make_async_remote_copy(
                    src_ref=acc_ref.at[r, pl.ds(peers[j], 1)],
                    dst_ref=stage_ref.at[r, pl.ds(j, 1)],
                    send_sem=send_sems.at[r, 0, j],
                    recv_sem=recv_sems.at[r, 0, j],
                    device_id=(peers[j],),
                    device_id_type=pl.DeviceIdType.MESH,
                ).wait_recv()
            red = (acc_ref[r, pl.ds(my, 1)]
                   + (stage_ref[r, 0:1] + stage_ref[r, 1:2])
                   + stage_ref[r, 2:3])
            fin_ref[r, pl.ds(my, 1)] = red

        def phase2_start(r):
            ds = []
            for j, p in enumerate(peers):
                d = pltpu.make_async_remote_copy(
                    src_ref=fin_ref.at[r, pl.ds(my, 1)],
                    dst_ref=fin_ref.at[r, pl.ds(my, 1)],
                    send_sem=send_sems.at[r, 1, j],
                    recv_sem=recv_sems.at[r, 1, j],
                    device_id=(p,),
                    device_id_type=pl.DeviceIdType.MESH,
                )
                d.start()
                ds.append(d)
            return ds

        def phase2_wait(r):
            for j in range(3):
                pltpu.make_async_remote_copy(
                    src_ref=fin_ref.at[r, pl.ds(my, 1)],
                    dst_ref=fin_ref.at[r, pl.ds(peers[j], 1)],
                    send_sem=send_sems.at[r, 1, j],
                    recv_sem=recv_sems.at[r, 1, j],
                    device_id=(peers[j],),
                    device_id_type=pl.DeviceIdType.MESH,
                ).wait_recv()

        for h in range(HQ_SHARD):
            attn_head(0, h)
        finish_half(0)
        p1s_0 = phase1_start(0)
        for h in range(6):
            attn_head(1, h)
        phase1_reduce(0, p1s_0)
        p2s_0 = phase2_start(0)
        for h in range(6, HQ_SHARD):
            attn_head(1, h)
        finish_half(1)
        p1s_1 = phase1_start(1)
        phase1_reduce(1, p1s_1)
        p2s_1 = phase2_start(1)
        phase2_wait(0)
        phase2_wait(1)
        for d in p1s_0 + p1s_1 + p2s_0 + p2s_1:
            d.wait_send()

        for r in range(2):
            for qq in range(N_DEV):
                out_ref[0, r * SQH:(r + 1) * SQH,
                        qq * QC:(qq + 1) * QC] = fin_ref[r, qq]

    grid_spec = pltpu.PrefetchScalarGridSpec(
        num_scalar_prefetch=1,
        grid=(1,),
        in_specs=[
            pl.BlockSpec((1, SQ, D), lambda i, m: (0, 0, 0)),
            pl.BlockSpec((D, D), lambda i, m: (0, 0)),
            pl.BlockSpec((D, D), lambda i, m: (0, 0)),
            pl.BlockSpec(memory_space=pl.ANY),
            pl.BlockSpec(memory_space=pl.ANY),
        ],
        out_specs=pl.BlockSpec((1, SQ, D), lambda i, m: (0, 0, 0)),
        scratch_shapes=[
            pltpu.VMEM((KV_SHARD, SKV, DH), jnp.float32),
            pltpu.VMEM((KV_SHARD, SKV, DH), jnp.float32),
            pltpu.VMEM((SQ, D), jnp.float32),
            pltpu.VMEM((2, N_DEV, SQH, QC), jnp.float32),
            pltpu.VMEM((2, 3, SQH, QC), jnp.float32),
            pltpu.VMEM((2, N_DEV, SQH, QC), jnp.float32),
            pltpu.SemaphoreType.DMA((4,)),
            pltpu.SemaphoreType.DMA((2, 2, 3)),
            pltpu.SemaphoreType.DMA((2, 2, 3)),
        ],
    )
    return pl.pallas_call(
        body,
        grid_spec=grid_spec,
        out_shape=jax.ShapeDtypeStruct((1, SQ, D), jnp.float32),
        compiler_params=pltpu.CompilerParams(collective_id=0),
    )(my_pos.reshape(1), x, Wq, Wo, K_ext, V_ext)
